# baseline (speedup 1.0000x reference)
import functools

import jax
import jax.numpy as jnp
from jax import lax
from jax.experimental import pallas as pl
from jax.experimental.pallas import tpu as pltpu
from jax.experimental.pallas import tpu_sc as plsc

E, D, H1, H2 = 8, 512, 256, 128
NC, NS, L = 2, 16, 16
NW = NC * NS
CH = 64
PHASES = ((16384, 512), (49152, 1024))

_SC_MESH = functools.partial(
    plsc.VectorSubcoreMesh, core_axis_name="c", subcore_axis_name="s")
_SC_PARAMS = pltpu.CompilerParams(needs_layout_passes=False)


def _celu(x):
    return jnp.where(x > 0, x, 0.1 * (jnp.exp(x / 0.1) - 1.0))


def _routing(species_h, n, tile):
    npad = n + E * tile
    nt = npad // tile
    cpw = n // NW
    spw = species_h.reshape(NW, cpw)
    cnt = jnp.sum(
        (spw[:, :, None] == jnp.arange(E, dtype=jnp.int32)[None, None, :])
        .astype(jnp.int32), axis=1)
    tot = jnp.sum(cnt, axis=0)
    padded = ((tot + tile - 1) // tile) * tile
    starts = (jnp.cumsum(padded) - padded).astype(jnp.int32)
    offs = starts[None, :] + (jnp.cumsum(cnt, axis=0) - cnt)
    offs_splat = jnp.broadcast_to(
        offs.astype(jnp.int32)[:, :, None], (NW, E, L))
    tile_starts = jnp.arange(nt, dtype=jnp.int32) * tile
    tile_expert = jnp.sum(
        (tile_starts[:, None] >= starts[None, :]).astype(jnp.int32),
        axis=1) - 1
    tile_expert = jnp.clip(tile_expert, 0, E - 1).astype(jnp.int32)
    return offs_splat, tile_expert, npad, nt


def _make_dispatch(n, npad, atom_base):
    cpw = n // NW
    nch = cpw // CH

    vpr = CH // L

    @functools.partial(
        pl.kernel,
        out_type=(jax.ShapeDtypeStruct((npad, D), jnp.float32),
                  jax.ShapeDtypeStruct((NW, nch, CH), jnp.int32)),
        mesh=_SC_MESH(),
        scratch_types=[
            pltpu.VMEM((cpw,), jnp.int32),
            pltpu.VMEM((E, L), jnp.int32),
            pltpu.VMEM((nch, CH), jnp.int32),
            pltpu.VMEM((CH, D), jnp.float32),
            pltpu.VMEM((CH, D), jnp.float32),
            pltpu.SemaphoreType.DMA,
            pltpu.SemaphoreType.DMA,
            pltpu.SemaphoreType.DMA,
            pltpu.SemaphoreType.DMA,
        ],
        compiler_params=_SC_PARAMS,
    )
    def dispatch(aev_hbm, spec_hbm, offs_hbm, gathered_hbm, dest_hbm,
                 spec_v, offs_v, idx_v, rows0, rows1,
                 semr0, semr1, semw0, semw1):
        wid = lax.axis_index("s") * NC + lax.axis_index("c")
        base = atom_base + wid * cpw
        pltpu.sync_copy(spec_hbm.at[wid], spec_v)
        pltpu.sync_copy(offs_hbm.at[wid], offs_v)

        def row_body(r, offs):
            for v in range(vpr):
                sv = spec_v[pl.ds((r * vpr + v) * L, L)]
                dst = jnp.zeros((L,), jnp.int32)
                new_offs = []
                for e in range(E):
                    m = sv == e
                    c = plsc.cumsum(jnp.where(m, 1, 0).astype(jnp.int32))
                    dst = jnp.where(m, offs[e] + c - 1, dst)
                    new_offs.append(
                        offs[e] + plsc.all_reduce_population_count(m))
                offs = tuple(new_offs)
                idx_v[r, pl.ds(v * L, L)] = dst
            return offs

        offs0 = tuple(offs_v[e, :] for e in range(E))
        lax.fori_loop(0, nch, row_body, offs0, unroll=False)
        pltpu.sync_copy(idx_v, dest_hbm.at[wid])

        rows, semr, semw = (rows0, rows1), (semr0, semr1), (semw0, semw1)

        def rd(i, b):
            return pltpu.make_async_copy(
                aev_hbm.at[pl.ds(base + i * CH, CH)], rows[b], semr[b])

        def wr(i, b):
            return pltpu.make_async_copy(
                rows[b], gathered_hbm.at[idx_v.at[i]], semw[b])

        rd(0, 0).start()
        rd(1, 1).start()

        def body(g, _):
            i0 = g * 2
            for b in range(2):
                i = i0 + b
                rd(i, b).wait()
                wr(i, b).start()
                wr(i, b).wait()

                @pl.when(i + 2 < nch)
                def _():
                    rd(i + 2, b).start()
            return 0

        lax.fori_loop(0, nch // 2, body, 0, unroll=False)

    return dispatch


def _make_mlp(npad, nt, tile):
    def body(te_ref, x_ref, w1_ref, b1_ref, w2_ref, b2_ref, w3_ref, b3_ref,
             o_ref):
        del te_ref
        x = x_ref[...]
        h = jnp.dot(x, w1_ref[0], preferred_element_type=jnp.float32)
        h = _celu(h + b1_ref[0])
        h = jnp.dot(h, w2_ref[0], preferred_element_type=jnp.float32)
        h = _celu(h + b2_ref[0])
        o = jnp.dot(h, w3_ref[0], preferred_element_type=jnp.float32)
        o_ref[...] = o + b3_ref[0]

    grid_spec = pltpu.PrefetchScalarGridSpec(
        num_scalar_prefetch=1,
        grid=(nt,),
        in_specs=[
            pl.BlockSpec((tile, D), lambda i, te: (i, 0)),
            pl.BlockSpec((1, D, H1), lambda i, te: (te[i], 0, 0)),
            pl.BlockSpec((1, 1, H1), lambda i, te: (te[i], 0, 0)),
            pl.BlockSpec((1, H1, H2), lambda i, te: (te[i], 0, 0)),
            pl.BlockSpec((1, 1, H2), lambda i, te: (te[i], 0, 0)),
            pl.BlockSpec((1, H2, 1), lambda i, te: (te[i], 0, 0)),
            pl.BlockSpec((1, 1, 1), lambda i, te: (te[i], 0, 0)),
        ],
        out_specs=pl.BlockSpec((tile, 1), lambda i, te: (i, 0)),
    )
    return pl.pallas_call(
        body,
        grid_spec=grid_spec,
        out_shape=jax.ShapeDtypeStruct((npad, 1), jnp.float32),
    )


def _make_collect(n, npad, n_mol, atoms_per_mol):
    cpw = n // NW
    mpw = n_mol // NW
    groups = mpw // L

    @functools.partial(
        pl.kernel,
        out_type=jax.ShapeDtypeStruct((n_mol,), jnp.float32),
        mesh=_SC_MESH(),
        scratch_types=[
            pltpu.VMEM((npad,), jnp.float32),
            pltpu.VMEM((cpw,), jnp.int32),
            pltpu.VMEM((mpw,), jnp.float32),
        ],
        compiler_params=_SC_PARAMS,
    )
    def collect(osort_hbm, dest_hbm, energy_hbm, vals_v, idx_v, out_v):
        wid = lax.axis_index("s") * NC + lax.axis_index("c")
        pltpu.sync_copy(osort_hbm, vals_v)
        pltpu.sync_copy(dest_hbm.at[wid], idx_v)
        lanes = lax.iota(jnp.int32, L)
        for g in range(groups):
            pos0 = g * L * atoms_per_mol + lanes * atoms_per_mol

            def body(k, acc):
                d = plsc.load_gather(idx_v, [pos0 + k])
                v = plsc.load_gather(vals_v, [d])
                return acc + v

            acc = lax.fori_loop(0, atoms_per_mol, body,
                                jnp.zeros((L,), jnp.float32), unroll=False)
            out_v[pl.ds(g * L, L)] = acc
        pltpu.sync_copy(out_v, energy_hbm.at[pl.ds(wid * mpw, mpw)])

    return collect


def kernel(species, aev, W1, b1, W2, b2, W3, b3):
    b_mol, a_per_mol = species.shape
    n = b_mol * a_per_mol
    species_flat = species.reshape(-1).astype(jnp.int32)
    aev_flat = aev.reshape(n, D)

    energies = []
    atom_base = 0
    for n_h, tile in PHASES:
        sp_h = lax.dynamic_slice_in_dim(species_flat, atom_base, n_h)
        offs, tile_expert, npad, nt = _routing(sp_h, n_h, tile)
        cpw = n_h // NW
        gathered, dest = _make_dispatch(n_h, npad, atom_base)(
            aev_flat, sp_h.reshape(NW, cpw), offs)
        out_sorted = _make_mlp(npad, nt, tile)(
            tile_expert, gathered,
            W1, b1.reshape(E, 1, H1), W2, b2.reshape(E, 1, H2),
            W3, b3.reshape(E, 1, 1))
        energies.append(
            _make_collect(n_h, npad, n_h // a_per_mol, a_per_mol)(
                out_sorted.reshape(npad), dest.reshape(NW, cpw)))
        atom_base += n_h

    return (species, jnp.concatenate(energies))


# speedup vs baseline: 1.2176x; 1.2176x over previous
import functools

import jax
import jax.numpy as jnp
from jax import lax
from jax.experimental import pallas as pl
from jax.experimental.pallas import tpu as pltpu
from jax.experimental.pallas import tpu_sc as plsc

E, D, H1, H2 = 8, 512, 256, 128
NC, NS, L = 2, 16, 16
NW = NC * NS
CH = 64
PHASES = ((16384, 512), (49152, 1024))

_SC_MESH = functools.partial(
    plsc.VectorSubcoreMesh, core_axis_name="c", subcore_axis_name="s")
_SC_PARAMS = pltpu.CompilerParams(needs_layout_passes=False)


def _celu(x):
    return jnp.where(x > 0, x, 0.1 * (jnp.exp(x / 0.1) - 1.0))


def _routing(species_h, n, tile):
    npad = n + E * tile
    nt = npad // tile
    cpw = n // NW
    spw = species_h.reshape(NW, cpw)
    cnt = jnp.sum(
        (spw[:, :, None] == jnp.arange(E, dtype=jnp.int32)[None, None, :])
        .astype(jnp.int32), axis=1)
    tot = jnp.sum(cnt, axis=0)
    padded = ((tot + tile - 1) // tile) * tile
    starts = (jnp.cumsum(padded) - padded).astype(jnp.int32)
    offs = starts[None, :] + (jnp.cumsum(cnt, axis=0) - cnt)
    offs_splat = jnp.broadcast_to(
        offs.astype(jnp.int32)[:, :, None], (NW, E, L))
    tile_starts = jnp.arange(nt, dtype=jnp.int32) * tile
    tile_expert = jnp.sum(
        (tile_starts[:, None] >= starts[None, :]).astype(jnp.int32),
        axis=1) - 1
    tile_expert = jnp.clip(tile_expert, 0, E - 1).astype(jnp.int32)
    return offs_splat, tile_expert, npad, nt


def _make_dispatch(n, npad, atom_base):
    cpw = n // NW
    nch = cpw // CH

    vpr = CH // L

    @functools.partial(
        pl.kernel,
        out_type=(jax.ShapeDtypeStruct((npad, D), jnp.float32),
                  jax.ShapeDtypeStruct((NW, nch, CH), jnp.int32)),
        mesh=_SC_MESH(),
        scratch_types=[
            pltpu.VMEM((cpw,), jnp.int32),
            pltpu.VMEM((E, L), jnp.int32),
            pltpu.VMEM((nch, CH), jnp.int32),
            pltpu.VMEM((CH, D), jnp.float32),
            pltpu.VMEM((CH, D), jnp.float32),
            pltpu.SemaphoreType.DMA,
            pltpu.SemaphoreType.DMA,
            pltpu.SemaphoreType.DMA,
            pltpu.SemaphoreType.DMA,
        ],
        compiler_params=_SC_PARAMS,
    )
    def dispatch(aev_hbm, spec_hbm, offs_hbm, gathered_hbm, dest_hbm,
                 spec_v, offs_v, idx_v, rows0, rows1,
                 semr0, semr1, semw0, semw1):
        wid = lax.axis_index("s") * NC + lax.axis_index("c")
        base = atom_base + wid * cpw
        rows, semr, semw = (rows0, rows1), (semr0, semr1), (semw0, semw1)

        def rd(i, b):
            return pltpu.make_async_copy(
                aev_hbm.at[pl.ds(base + i * CH, CH)], rows[b], semr[b])

        def wr(i, b):
            return pltpu.make_async_copy(
                rows[b], gathered_hbm.at[idx_v.at[i]], semw[b])

        rd(0, 0).start()
        rd(1, 1).start()
        pltpu.sync_copy(spec_hbm.at[wid], spec_v)
        pltpu.sync_copy(offs_hbm.at[wid], offs_v)

        def row_body(r, offs):
            for v in range(vpr):
                sv = spec_v[pl.ds((r * vpr + v) * L, L)]
                dst = jnp.zeros((L,), jnp.int32)
                new_offs = []
                for e in range(E):
                    m = sv == e
                    c = plsc.cumsum(jnp.where(m, 1, 0).astype(jnp.int32))
                    dst = jnp.where(m, offs[e] + c - 1, dst)
                    new_offs.append(
                        offs[e] + plsc.all_reduce_population_count(m))
                offs = tuple(new_offs)
                idx_v[r, pl.ds(v * L, L)] = dst
            return offs

        offs0 = tuple(offs_v[e, :] for e in range(E))
        lax.fori_loop(0, nch, row_body, offs0, unroll=False)
        pltpu.sync_copy(idx_v, dest_hbm.at[wid])

        def body(g, _):
            i0 = g * 2
            for b in range(2):
                i = i0 + b
                rd(i, b).wait()
                wr(i, b).start()
                wr(i, b).wait()

                @pl.when(i + 2 < nch)
                def _():
                    rd(i + 2, b).start()
            return 0

        lax.fori_loop(0, nch // 2, body, 0, unroll=False)

    return dispatch


def _make_mlp(npad, nt, tile):
    def body(te_ref, x_ref, w1_ref, b1_ref, w2_ref, b2_ref, w3_ref, b3_ref,
             o_ref):
        del te_ref
        x = x_ref[...]
        h = jnp.dot(x, w1_ref[0], preferred_element_type=jnp.float32)
        h = _celu(h + b1_ref[0])
        h = jnp.dot(h, w2_ref[0], preferred_element_type=jnp.float32)
        h = _celu(h + b2_ref[0])
        o = lax.dot_general(w3_ref[0], h, (((1,), (1,)), ((), ())),
                            preferred_element_type=jnp.float32)
        o_ref[...] = o[0] + b3_ref[0, 0]

    grid_spec = pltpu.PrefetchScalarGridSpec(
        num_scalar_prefetch=1,
        grid=(nt,),
        in_specs=[
            pl.BlockSpec((tile, D), lambda i, te: (i, 0)),
            pl.BlockSpec((1, D, H1), lambda i, te: (te[i], 0, 0)),
            pl.BlockSpec((1, 1, H1), lambda i, te: (te[i], 0, 0)),
            pl.BlockSpec((1, H1, H2), lambda i, te: (te[i], 0, 0)),
            pl.BlockSpec((1, 1, H2), lambda i, te: (te[i], 0, 0)),
            pl.BlockSpec((1, 1, H2), lambda i, te: (te[i], 0, 0)),
            pl.BlockSpec((1, 1, 1), lambda i, te: (te[i], 0, 0)),
        ],
        out_specs=pl.BlockSpec((tile,), lambda i, te: (i,)),
    )
    return pl.pallas_call(
        body,
        grid_spec=grid_spec,
        out_shape=jax.ShapeDtypeStruct((npad,), jnp.float32),
    )


def _make_collect(n, npad, n_mol, atoms_per_mol):
    cpw = n // NW
    mpw = n_mol // NW
    groups = mpw // L

    @functools.partial(
        pl.kernel,
        out_type=jax.ShapeDtypeStruct((n_mol,), jnp.float32),
        mesh=_SC_MESH(),
        scratch_types=[
            pltpu.VMEM((npad,), jnp.float32),
            pltpu.VMEM((cpw,), jnp.int32),
            pltpu.VMEM((mpw,), jnp.float32),
        ],
        compiler_params=_SC_PARAMS,
    )
    def collect(osort_hbm, dest_hbm, energy_hbm, vals_v, idx_v, out_v):
        wid = lax.axis_index("s") * NC + lax.axis_index("c")
        pltpu.sync_copy(osort_hbm, vals_v)
        pltpu.sync_copy(dest_hbm.at[wid], idx_v)
        lanes = lax.iota(jnp.int32, L)
        for g in range(groups):
            pos0 = g * L * atoms_per_mol + lanes * atoms_per_mol

            def body(k, acc):
                d = plsc.load_gather(idx_v, [pos0 + k])
                v = plsc.load_gather(vals_v, [d])
                return acc + v

            acc = lax.fori_loop(0, atoms_per_mol, body,
                                jnp.zeros((L,), jnp.float32), unroll=False)
            out_v[pl.ds(g * L, L)] = acc
        pltpu.sync_copy(out_v, energy_hbm.at[pl.ds(wid * mpw, mpw)])

    return collect


def kernel(species, aev, W1, b1, W2, b2, W3, b3):
    b_mol, a_per_mol = species.shape
    n = b_mol * a_per_mol
    species_flat = species.reshape(-1).astype(jnp.int32)
    aev_flat = aev.reshape(n, D)

    energies = []
    atom_base = 0
    for n_h, tile in PHASES:
        sp_h = lax.dynamic_slice_in_dim(species_flat, atom_base, n_h)
        offs, tile_expert, npad, nt = _routing(sp_h, n_h, tile)
        cpw = n_h // NW
        gathered, dest = _make_dispatch(n_h, npad, atom_base)(
            aev_flat, sp_h.reshape(NW, cpw), offs)
        out_sorted = _make_mlp(npad, nt, tile)(
            tile_expert, gathered,
            W1, b1.reshape(E, 1, H1), W2, b2.reshape(E, 1, H2),
            W3.reshape(E, 1, H2), b3.reshape(E, 1, 1))
        energies.append(
            _make_collect(n_h, npad, n_h // a_per_mol, a_per_mol)(
                out_sorted.reshape(npad), dest.reshape(NW, cpw)))
        atom_base += n_h

    return (species, jnp.concatenate(energies))
